# baseline (speedup 1.0000x reference)
import functools

import jax
import jax.numpy as jnp
from jax import lax
from jax.experimental import pallas as pl
from jax.experimental.pallas import tpu as pltpu
from jax.experimental.pallas import tpu_sc as plsc

B, P, K = 8, 2048, 16
C = 32
NUM_RBF = 32
CUTOFF = 5.0
NUM_LAYERS = 4
RH = 64
NUM_CLASSES = 10

BP = B * P
BE = BP * K
RKNN = 256
RNODE = 256
EBLK = RNODE * K


def _knn_body(pos_ref, post_ref, nbr_ref, *, b_base):
    b = pl.program_id(0) + b_base
    i = pl.program_id(1)
    x = pos_ref[0]
    xt = post_ref[0]
    d2 = jnp.zeros((RKNN, P), jnp.float32)
    for d in range(3):
        diff = x[:, d:d + 1] - xt[d:d + 1, :]
        d2 = d2 + diff * diff
    rows = lax.broadcasted_iota(jnp.int32, (RKNN, P), 0) + i * RKNN
    cols = lax.broadcasted_iota(jnp.int32, (RKNN, P), 1)
    d2 = jnp.where(rows == cols, d2 + 1e9, d2)
    idx_cols = []
    for _ in range(K):
        m = jnp.min(d2, axis=1, keepdims=True)
        cand = jnp.where(d2 == m, cols, jnp.int32(P))
        idx = jnp.min(cand, axis=1, keepdims=True)
        idx_cols.append(idx)
        d2 = jnp.where(cols == idx, jnp.float32(3e38), d2)
    nbr = jnp.concatenate(idx_cols, axis=1)
    nbr_ref[0] = nbr + b * P


def _run_knn(batch_h, b_base):
    nb = batch_h.shape[0]
    post = jnp.transpose(batch_h, (0, 2, 1))
    return pl.pallas_call(
        functools.partial(_knn_body, b_base=b_base),
        grid=(nb, P // RKNN),
        in_specs=[
            pl.BlockSpec((1, RKNN, 3), lambda b, i: (b, i, 0)),
            pl.BlockSpec((1, 3, P), lambda b, i: (b, 0, 0)),
        ],
        out_specs=pl.BlockSpec((1, RKNN, K), lambda b, i: (b, i, 0)),
        out_shape=jax.ShapeDtypeStruct((nb, P, K), jnp.int32),
    )(batch_h, post)


_SC_NC = 2
_SC_NS = 16
_NW = _SC_NC * _SC_NS


def _sc_gather(table, idx, chunk):
    n = idx.shape[0]
    d = table.shape[1]
    dt = table.dtype
    npw = n // _NW
    nchunks = npw // chunk
    mesh = plsc.VectorSubcoreMesh(core_axis_name="c", subcore_axis_name="s")

    @functools.partial(
        pl.kernel,
        mesh=mesh,
        out_type=jax.ShapeDtypeStruct((n, d), dt),
        scratch_types=[
            pltpu.VMEM((chunk,), jnp.int32),
            pltpu.VMEM((chunk, d), dt),
            pltpu.SemaphoreType.DMA,
        ],
    )
    def gather_k(table_hbm, idx_hbm, out_hbm, idx_v, rows_v, sem):
        wid = lax.axis_index("s") * _SC_NC + lax.axis_index("c")
        base = wid * npw

        def body(ci, carry):
            off = base + ci * chunk
            pltpu.sync_copy(idx_hbm.at[pl.ds(off, chunk)], idx_v)
            pltpu.async_copy(table_hbm.at[idx_v], rows_v, sem).wait()
            pltpu.sync_copy(rows_v, out_hbm.at[pl.ds(off, chunk)])
            return carry

        lax.fori_loop(0, nchunks, body, 0)

    return gather_k(table, idx)


def _edge_body(ps_ref, pd_ref, centers_ref, rbf_ref, rh_ref):
    ps = ps_ref[...]
    pd = pd_ref[...]
    relx = ps[:, 0:1] - pd[:, 0:1]
    rely = ps[:, 1:2] - pd[:, 1:2]
    relz = ps[:, 2:3] - pd[:, 2:3]
    r = jnp.sqrt(relx * relx + rely * rely + relz * relz + 1e-12)
    rhx = relx / r
    rhy = rely / r
    rhz = relz / r
    centers = centers_ref[...]
    gamma = jnp.float32(NUM_RBF / CUTOFF)
    t = r - centers
    rbf = jnp.exp(-gamma * (t * t))
    x = jnp.clip(r * jnp.float32(1.0 / CUTOFF), 0.0, 1.0)
    u = jnp.float32(jnp.pi) * (x - 0.5)
    u2 = u * u
    sinu = u * (1.0 + u2 * (jnp.float32(-1.0 / 6.0)
                + u2 * (jnp.float32(1.0 / 120.0)
                + u2 * (jnp.float32(-1.0 / 5040.0)
                + u2 * jnp.float32(1.0 / 362880.0)))))
    env = jnp.where(x >= 1.0, 0.0, 0.5 * (1.0 - sinu))
    rbf_ref[...] = rbf * env
    one = jnp.ones_like(rhx)
    rh_ref[...] = jnp.concatenate(
        [rhx, rhy, rhz, one, jnp.zeros((EBLK, 4), jnp.float32)], axis=1)


def _run_edges(pos_src, pos_dst, centers):
    ne = pos_src.shape[0]
    return pl.pallas_call(
        _edge_body,
        grid=(ne // EBLK,),
        in_specs=[
            pl.BlockSpec((EBLK, 128), lambda g: (g, 0)),
            pl.BlockSpec((EBLK, 4), lambda g: (g, 0)),
            pl.BlockSpec((1, NUM_RBF), lambda g: (0, 0)),
        ],
        out_specs=[
            pl.BlockSpec((EBLK, NUM_RBF), lambda g: (g, 0)),
            pl.BlockSpec((EBLK, 8), lambda g: (g, 0)),
        ],
        out_shape=[
            jax.ShapeDtypeStruct((ne, NUM_RBF), jnp.float32),
            jax.ShapeDtypeStruct((ne, 8), jnp.float32),
        ],
    )(pos_src, pos_dst, centers)


def _silu(x):
    return x * (1.0 / (1.0 + jnp.exp(-x)))


def _layer_body(g_ref, rbf_ref, rh_ref, old_ref,
                w1_ref, b1_ref, w2a_ref, b2a_ref, w2b_ref, b2b_ref,
                sel3_ref, self1_ref, wg_ref, bg_ref, wbig_ref,
                mrow1_ref, mrow02_ref, mrow3_ref, out_ref, *, layer0):
    rbf = rbf_ref[...]
    h = _silu(jnp.dot(rbf.astype(jnp.bfloat16), w1_ref[...],
                      preferred_element_type=jnp.float32) + b1_ref[...])
    hb = h.astype(jnp.bfloat16)
    a = jnp.dot(hb, w2a_ref[...],
                preferred_element_type=jnp.float32) + b2a_ref[...]
    rh = rh_ref[...]
    r3 = jnp.dot(rh, sel3_ref[...], preferred_element_type=jnp.float32)
    rfull = jnp.dot(rh, self1_ref[...], preferred_element_type=jnp.float32)
    m1 = mrow1_ref[...] != 0.0
    m02 = mrow02_ref[...] != 0.0
    m3 = mrow3_ref[...] != 0.0
    if layer0:
        s4 = jnp.broadcast_to(g_ref[...], (EBLK, 4 * C))
        msg = a * s4 * rfull
    else:
        g = g_ref[...].astype(jnp.float32)
        bt = jnp.dot(hb, w2b_ref[...],
                     preferred_element_type=jnp.float32) + b2b_ref[...]
        t = g * r3
        t2 = t + jnp.roll(t, -C, axis=1)
        t4 = t2 + jnp.roll(t2, -2 * C, axis=1)
        ghat = jnp.where(m3, t4, g)
        x1 = jnp.where(m1, jnp.roll(g, -2 * C, axis=1), g)
        s4 = jnp.where(m02, jnp.roll(x1, -C, axis=1), x1)
        msg = a * s4 * rfull + bt * ghat
    agg = jnp.sum(msg.reshape(RNODE, K, 4 * C), axis=1) * jnp.float32(1.0 / K)
    a_s = agg[:, 3 * C:4 * C]
    gate = 1.0 / (1.0 + jnp.exp(-(jnp.dot(a_s, wg_ref[...],
                                          preferred_element_type=jnp.float32)
                                  + bg_ref[...])))
    x = jnp.concatenate(
        [agg[:, 0:C] * gate, agg[:, C:2 * C] * gate, agg[:, 2 * C:3 * C] * gate,
         _silu(a_s)], axis=1)
    z = jnp.dot(x, wbig_ref[...], preferred_element_type=jnp.float32)
    out_ref[...] = old_ref[...] + z


def _run_layer(gathered, rbf, rh, sv_old, w1, b1, w2a, b2a, w2b, b2b,
               sel3, self1, wg, bg, wbig, mrows, layer0=False):
    full = lambda r, c: pl.BlockSpec((r, c), lambda g: (0, 0))
    g_spec = (full(1, 4 * C) if layer0
              else pl.BlockSpec((EBLK, 4 * C), lambda g: (g, 0)))
    nn = sv_old.shape[0]
    return pl.pallas_call(
        functools.partial(_layer_body, layer0=layer0),
        grid=(nn // RNODE,),
        in_specs=[
            g_spec,
            pl.BlockSpec((EBLK, NUM_RBF), lambda g: (g, 0)),
            pl.BlockSpec((EBLK, 8), lambda g: (g, 0)),
            pl.BlockSpec((RNODE, 4 * C), lambda g: (g, 0)),
            full(NUM_RBF, RH), full(1, RH),
            full(RH, 4 * C), full(1, 4 * C), full(RH, 4 * C), full(1, 4 * C),
            full(8, 4 * C), full(8, 4 * C),
            full(C, C), full(1, C), full(4 * C, 4 * C),
            full(1, 4 * C), full(1, 4 * C), full(1, 4 * C),
        ],
        out_specs=pl.BlockSpec((RNODE, 4 * C), lambda g: (g, 0)),
        out_shape=jax.ShapeDtypeStruct((nn, 4 * C), jnp.float32),
    )(gathered, rbf, rh, sv_old, w1, b1, w2a, b2a, w2b, b2b,
      sel3, self1, wg, bg, wbig, mrows[0], mrows[1], mrows[2])


def _readout_body(sv_ref, wc1_ref, bc1_ref, wc2_ref, bc2_ref, wc3_ref, bc3_ref,
                  out_ref):
    s = sv_ref[:, 3 * C:4 * C]
    pooled = jnp.sum(s.reshape(B, P, C), axis=1) * jnp.float32(1.0 / P)
    h = _silu(jnp.dot(pooled, wc1_ref[...],
                      preferred_element_type=jnp.float32) + bc1_ref[...])
    h = _silu(jnp.dot(h, wc2_ref[...],
                      preferred_element_type=jnp.float32) + bc2_ref[...])
    out_ref[...] = jnp.dot(h, wc3_ref[...],
                           preferred_element_type=jnp.float32) + bc3_ref[...]


def _run_readout(sv, wc1, bc1, wc2, bc2, wc3, bc3):
    return pl.pallas_call(
        _readout_body,
        out_shape=jax.ShapeDtypeStruct((B, NUM_CLASSES), jnp.float32),
    )(sv, wc1, bc1, wc2, bc2, wc3, bc3)


def kernel(batch, embed_w, W1, b1, W2, b2, Wg, bg, Wms, Wmv,
           Wc1, bc1, Wc2, bc2, Wc3, bc3):
    batch = batch.astype(jnp.float32)
    NH = 8
    BH = B // NH
    BPH, BEH = BH * P, BH * P * K

    pos_flat = batch.reshape(BP, 3)
    pos_pad = jnp.concatenate(
        [pos_flat, jnp.zeros((BP, 125), jnp.float32)], axis=1)
    centers = jnp.linspace(0.0, CUTOFF, NUM_RBF,
                           dtype=jnp.float32).reshape(1, NUM_RBF)

    flat_idx, rbf, rh = [], [], []
    for h in range(NH):
        nbr_h = _run_knn(batch[h * BH:(h + 1) * BH], h * BH)
        flat_idx.append(nbr_h.reshape(BEH))
    for h in range(NH):
        pos_src_h = _sc_gather(pos_pad, flat_idx[h], chunk=512)
        pd_h = pos_flat[h * BPH:(h + 1) * BPH]
        pos_dst_h = jnp.broadcast_to(
            pd_h[:, None, :], (BPH, K, 3)).reshape(BEH, 3)
        pos_dst_h = jnp.concatenate(
            [pos_dst_h, jnp.zeros((BEH, 1), jnp.float32)], axis=1)
        rbf_h, rh_h = _run_edges(pos_src_h, pos_dst_h, centers)
        rbf.append(rbf_h)
        rh.append(rh_h)

    embed = embed_w.astype(jnp.float32)
    sv = [jnp.concatenate(
        [jnp.zeros((BPH, 3 * C), jnp.float32),
         jnp.broadcast_to(embed, (BPH, C))], axis=1) for _ in range(NH)]

    lane = jnp.arange(4 * C, dtype=jnp.int32)
    plane = lane // C
    mrow1 = (plane == 1).astype(jnp.float32).reshape(1, 4 * C)
    mrow02 = ((plane == 0) | (plane == 2)).astype(jnp.float32).reshape(1, 4 * C)
    mrow3 = (plane == 3).astype(jnp.float32).reshape(1, 4 * C)
    mrows = (mrow1, mrow02, mrow3)
    sel3 = jnp.zeros((8, 4 * C), jnp.float32)
    for d in range(3):
        sel3 = sel3.at[d, d * C:(d + 1) * C].set(1.0)
    self1 = sel3.at[3, 3 * C:4 * C].set(1.0)

    for l in range(NUM_LAYERS):
        w2l = W2[l]
        b2l = b2[l]
        w2a = jnp.concatenate([w2l[:, 2 * C:3 * C]] * 3 + [w2l[:, 0:C]], axis=1)
        b2a = jnp.concatenate([b2l[2 * C:3 * C]] * 3
                              + [b2l[0:C]]).reshape(1, 4 * C)
        w2b = jnp.concatenate([w2l[:, 3 * C:4 * C]] * 3
                              + [w2l[:, C:2 * C]], axis=1)
        b2b = jnp.concatenate([b2l[3 * C:4 * C]] * 3
                              + [b2l[C:2 * C]]).reshape(1, 4 * C)
        wbig = jnp.zeros((4 * C, 4 * C), jnp.float32)
        for d in range(3):
            wbig = wbig.at[d * C:(d + 1) * C, d * C:(d + 1) * C].set(Wmv[l])
        wbig = wbig.at[3 * C:4 * C, 3 * C:4 * C].set(Wms[l])
        sv_full = None if l == 0 else jnp.concatenate(sv, axis=0)
        new_sv = []
        for h in range(NH):
            if l == 0:
                gathered = jnp.tile(embed, (1, 4))
            else:
                gathered = _sc_gather(sv_full, flat_idx[h], chunk=512)
            o_f32 = _run_layer(
                gathered, rbf[h], rh[h], sv[h],
                W1[l].astype(jnp.bfloat16), b1[l].reshape(1, RH),
                w2a.astype(jnp.bfloat16), b2a,
                w2b.astype(jnp.bfloat16), b2b, sel3, self1,
                Wg[l], bg[l].reshape(1, C),
                wbig, mrows, layer0=(l == 0))
            new_sv.append(o_f32)
        sv = new_sv

    return _run_readout(jnp.concatenate(sv, axis=0),
                        Wc1, bc1.reshape(1, 128),
                        Wc2, bc2.reshape(1, 64),
                        Wc3, bc3.reshape(1, NUM_CLASSES))


# speedup vs baseline: 1.0475x; 1.0475x over previous
import functools

import jax
import jax.numpy as jnp
from jax import lax
from jax.experimental import pallas as pl
from jax.experimental.pallas import tpu as pltpu
from jax.experimental.pallas import tpu_sc as plsc

B, P, K = 8, 2048, 16
C = 32
NUM_RBF = 32
CUTOFF = 5.0
NUM_LAYERS = 4
RH = 64
NUM_CLASSES = 10

BP = B * P
BE = BP * K
RKNN = 256
RNODE = 256
EBLK = RNODE * K


def _knn_body(pos_ref, post_ref, nbr_ref, *, b_base):
    b = pl.program_id(0) + b_base
    i = pl.program_id(1)
    x = pos_ref[0]
    xt = post_ref[0]
    d2 = jnp.zeros((RKNN, P), jnp.float32)
    for d in range(3):
        diff = x[:, d:d + 1] - xt[d:d + 1, :]
        d2 = d2 + diff * diff
    rows = lax.broadcasted_iota(jnp.int32, (RKNN, P), 0) + i * RKNN
    cols = lax.broadcasted_iota(jnp.int32, (RKNN, P), 1)
    d2 = jnp.where(rows == cols, d2 + 1e9, d2)
    idx_cols = []
    for _ in range(K):
        m = jnp.min(d2, axis=1, keepdims=True)
        cand = jnp.where(d2 == m, cols, jnp.int32(P))
        idx = jnp.min(cand, axis=1, keepdims=True)
        idx_cols.append(idx)
        d2 = jnp.where(cols == idx, jnp.float32(3e38), d2)
    nbr = jnp.concatenate(idx_cols, axis=1)
    nbr_ref[0] = nbr + b * P


def _run_knn(batch_h, b_base):
    nb = batch_h.shape[0]
    post = jnp.transpose(batch_h, (0, 2, 1))
    return pl.pallas_call(
        functools.partial(_knn_body, b_base=b_base),
        grid=(nb, P // RKNN),
        in_specs=[
            pl.BlockSpec((1, RKNN, 3), lambda b, i: (b, i, 0)),
            pl.BlockSpec((1, 3, P), lambda b, i: (b, 0, 0)),
        ],
        out_specs=pl.BlockSpec((1, RKNN, K), lambda b, i: (b, i, 0)),
        out_shape=jax.ShapeDtypeStruct((nb, P, K), jnp.int32),
    )(batch_h, post)


_SC_NC = 2
_SC_NS = 16
_NW = _SC_NC * _SC_NS


def _sc_gather(table, idx, chunk):
    n = idx.shape[0]
    d = table.shape[1]
    dt = table.dtype
    npw = n // _NW
    nchunks = npw // chunk
    mesh = plsc.VectorSubcoreMesh(core_axis_name="c", subcore_axis_name="s")

    @functools.partial(
        pl.kernel,
        mesh=mesh,
        out_type=jax.ShapeDtypeStruct((n, d), dt),
        scratch_types=[
            pltpu.VMEM((chunk,), jnp.int32),
            pltpu.VMEM((chunk, d), dt),
            pltpu.SemaphoreType.DMA,
        ],
    )
    def gather_k(table_hbm, idx_hbm, out_hbm, idx_v, rows_v, sem):
        wid = lax.axis_index("s") * _SC_NC + lax.axis_index("c")
        base = wid * npw

        def body(ci, carry):
            off = base + ci * chunk
            pltpu.sync_copy(idx_hbm.at[pl.ds(off, chunk)], idx_v)
            pltpu.async_copy(table_hbm.at[idx_v], rows_v, sem).wait()
            pltpu.sync_copy(rows_v, out_hbm.at[pl.ds(off, chunk)])
            return carry

        lax.fori_loop(0, nchunks, body, 0)

    return gather_k(table, idx)


def _edge_body(ps_ref, pd_ref, centers_ref, rbf_ref, rh_ref):
    ps = ps_ref[...]
    pd = pd_ref[...]
    relx = ps[:, 0:1] - pd[:, 0:1]
    rely = ps[:, 1:2] - pd[:, 1:2]
    relz = ps[:, 2:3] - pd[:, 2:3]
    r = jnp.sqrt(relx * relx + rely * rely + relz * relz + 1e-12)
    rhx = relx / r
    rhy = rely / r
    rhz = relz / r
    centers = centers_ref[...]
    gamma = jnp.float32(NUM_RBF / CUTOFF)
    t = r - centers
    rbf = jnp.exp(-gamma * (t * t))
    x = jnp.clip(r * jnp.float32(1.0 / CUTOFF), 0.0, 1.0)
    u = jnp.float32(jnp.pi) * (x - 0.5)
    u2 = u * u
    sinu = u * (1.0 + u2 * (jnp.float32(-1.0 / 6.0)
                + u2 * (jnp.float32(1.0 / 120.0)
                + u2 * (jnp.float32(-1.0 / 5040.0)
                + u2 * jnp.float32(1.0 / 362880.0)))))
    env = jnp.where(x >= 1.0, 0.0, 0.5 * (1.0 - sinu))
    rbf_ref[...] = rbf * env
    one = jnp.ones_like(rhx)
    rh_ref[...] = jnp.concatenate(
        [rhx, rhy, rhz, one, jnp.zeros((EBLK, 4), jnp.float32)], axis=1)


def _run_edges(pos_src, pos_dst, centers):
    ne = pos_src.shape[0]
    return pl.pallas_call(
        _edge_body,
        grid=(ne // EBLK,),
        in_specs=[
            pl.BlockSpec((EBLK, 128), lambda g: (g, 0)),
            pl.BlockSpec((EBLK, 4), lambda g: (g, 0)),
            pl.BlockSpec((1, NUM_RBF), lambda g: (0, 0)),
        ],
        out_specs=[
            pl.BlockSpec((EBLK, NUM_RBF), lambda g: (g, 0)),
            pl.BlockSpec((EBLK, 8), lambda g: (g, 0)),
        ],
        out_shape=[
            jax.ShapeDtypeStruct((ne, NUM_RBF), jnp.float32),
            jax.ShapeDtypeStruct((ne, 8), jnp.float32),
        ],
    )(pos_src, pos_dst, centers)


def _silu(x):
    return x * (1.0 / (1.0 + jnp.exp(-x)))


def _layer_body(g_ref, rbf_ref, rh_ref, old_ref,
                w1_ref, b1_ref, w2a_ref, b2a_ref, w2b_ref, b2b_ref,
                sel3_ref, self1_ref, wg_ref, bg_ref, wbig_ref,
                mrow1_ref, mrow02_ref, mrow3_ref, out_ref, *, layer0):
    rbf = rbf_ref[...]
    h = _silu(jnp.dot(rbf.astype(jnp.bfloat16), w1_ref[...],
                      preferred_element_type=jnp.float32) + b1_ref[...])
    hb = h.astype(jnp.bfloat16)
    a = jnp.dot(hb, w2a_ref[...],
                preferred_element_type=jnp.float32) + b2a_ref[...]
    rh = rh_ref[...]
    r3 = jnp.dot(rh, sel3_ref[...], preferred_element_type=jnp.float32)
    rfull = jnp.dot(rh, self1_ref[...], preferred_element_type=jnp.float32)
    m1 = mrow1_ref[...] != 0.0
    m02 = mrow02_ref[...] != 0.0
    m3 = mrow3_ref[...] != 0.0
    if layer0:
        s4 = jnp.broadcast_to(g_ref[...], (EBLK, 4 * C))
        msg = a * s4 * rfull
    else:
        g = g_ref[...].astype(jnp.float32)
        bt = jnp.dot(hb, w2b_ref[...],
                     preferred_element_type=jnp.float32) + b2b_ref[...]
        t = g * r3
        t2 = t + jnp.roll(t, -C, axis=1)
        t4 = t2 + jnp.roll(t2, -2 * C, axis=1)
        ghat = jnp.where(m3, t4, g)
        x1 = jnp.where(m1, jnp.roll(g, -2 * C, axis=1), g)
        s4 = jnp.where(m02, jnp.roll(x1, -C, axis=1), x1)
        msg = a * s4 * rfull + bt * ghat
    agg = jnp.sum(msg.reshape(RNODE, K, 4 * C), axis=1) * jnp.float32(1.0 / K)
    a_s = agg[:, 3 * C:4 * C]
    gate = 1.0 / (1.0 + jnp.exp(-(jnp.dot(a_s, wg_ref[...],
                                          preferred_element_type=jnp.float32)
                                  + bg_ref[...])))
    x = jnp.concatenate(
        [agg[:, 0:C] * gate, agg[:, C:2 * C] * gate, agg[:, 2 * C:3 * C] * gate,
         _silu(a_s)], axis=1)
    z = jnp.dot(x, wbig_ref[...], preferred_element_type=jnp.float32)
    out_ref[...] = old_ref[...] + z


def _run_layer(gathered, rbf, rh, sv_old, w1, b1, w2a, b2a, w2b, b2b,
               sel3, self1, wg, bg, wbig, mrows, layer0=False):
    full = lambda r, c: pl.BlockSpec((r, c), lambda g: (0, 0))
    g_spec = (full(1, 4 * C) if layer0
              else pl.BlockSpec((EBLK, 4 * C), lambda g: (g, 0)))
    nn = sv_old.shape[0]
    return pl.pallas_call(
        functools.partial(_layer_body, layer0=layer0),
        grid=(nn // RNODE,),
        in_specs=[
            g_spec,
            pl.BlockSpec((EBLK, NUM_RBF), lambda g: (g, 0)),
            pl.BlockSpec((EBLK, 8), lambda g: (g, 0)),
            pl.BlockSpec((RNODE, 4 * C), lambda g: (g, 0)),
            full(NUM_RBF, RH), full(1, RH),
            full(RH, 4 * C), full(1, 4 * C), full(RH, 4 * C), full(1, 4 * C),
            full(8, 4 * C), full(8, 4 * C),
            full(C, C), full(1, C), full(4 * C, 4 * C),
            full(1, 4 * C), full(1, 4 * C), full(1, 4 * C),
        ],
        out_specs=pl.BlockSpec((RNODE, 4 * C), lambda g: (g, 0)),
        out_shape=jax.ShapeDtypeStruct((nn, 4 * C), jnp.float32),
    )(gathered, rbf, rh, sv_old, w1, b1, w2a, b2a, w2b, b2b,
      sel3, self1, wg, bg, wbig, mrows[0], mrows[1], mrows[2])


def _readout_body(sv_ref, wc1_ref, bc1_ref, wc2_ref, bc2_ref, wc3_ref, bc3_ref,
                  out_ref):
    s = sv_ref[:, 3 * C:4 * C]
    pooled = jnp.sum(s.reshape(B, P, C), axis=1) * jnp.float32(1.0 / P)
    h = _silu(jnp.dot(pooled, wc1_ref[...],
                      preferred_element_type=jnp.float32) + bc1_ref[...])
    h = _silu(jnp.dot(h, wc2_ref[...],
                      preferred_element_type=jnp.float32) + bc2_ref[...])
    out_ref[...] = jnp.dot(h, wc3_ref[...],
                           preferred_element_type=jnp.float32) + bc3_ref[...]


def _run_readout(sv, wc1, bc1, wc2, bc2, wc3, bc3):
    return pl.pallas_call(
        _readout_body,
        out_shape=jax.ShapeDtypeStruct((B, NUM_CLASSES), jnp.float32),
    )(sv, wc1, bc1, wc2, bc2, wc3, bc3)


def kernel(batch, embed_w, W1, b1, W2, b2, Wg, bg, Wms, Wmv,
           Wc1, bc1, Wc2, bc2, Wc3, bc3):
    batch = batch.astype(jnp.float32)
    NH = 4
    BH = B // NH
    BPH, BEH = BH * P, BH * P * K

    pos_flat = batch.reshape(BP, 3)
    pos_pad = jnp.concatenate(
        [pos_flat, jnp.zeros((BP, 125), jnp.float32)], axis=1)
    centers = jnp.linspace(0.0, CUTOFF, NUM_RBF,
                           dtype=jnp.float32).reshape(1, NUM_RBF)

    flat_idx, rbf, rh = [], [], []
    for h in range(NH):
        nbr_h = _run_knn(batch[h * BH:(h + 1) * BH], h * BH)
        flat_idx.append(nbr_h.reshape(BEH))
    for h in range(NH):
        pos_src_h = _sc_gather(pos_pad, flat_idx[h], chunk=512)
        pd_h = pos_flat[h * BPH:(h + 1) * BPH]
        pos_dst_h = jnp.broadcast_to(
            pd_h[:, None, :], (BPH, K, 3)).reshape(BEH, 3)
        pos_dst_h = jnp.concatenate(
            [pos_dst_h, jnp.zeros((BEH, 1), jnp.float32)], axis=1)
        rbf_h, rh_h = _run_edges(pos_src_h, pos_dst_h, centers)
        rbf.append(rbf_h)
        rh.append(rh_h)

    embed = embed_w.astype(jnp.float32)
    sv = [jnp.concatenate(
        [jnp.zeros((BPH, 3 * C), jnp.float32),
         jnp.broadcast_to(embed, (BPH, C))], axis=1) for _ in range(NH)]

    lane = jnp.arange(4 * C, dtype=jnp.int32)
    plane = lane // C
    mrow1 = (plane == 1).astype(jnp.float32).reshape(1, 4 * C)
    mrow02 = ((plane == 0) | (plane == 2)).astype(jnp.float32).reshape(1, 4 * C)
    mrow3 = (plane == 3).astype(jnp.float32).reshape(1, 4 * C)
    mrows = (mrow1, mrow02, mrow3)
    sel3 = jnp.zeros((8, 4 * C), jnp.float32)
    for d in range(3):
        sel3 = sel3.at[d, d * C:(d + 1) * C].set(1.0)
    self1 = sel3.at[3, 3 * C:4 * C].set(1.0)

    for l in range(NUM_LAYERS):
        w2l = W2[l]
        b2l = b2[l]
        w2a = jnp.concatenate([w2l[:, 2 * C:3 * C]] * 3 + [w2l[:, 0:C]], axis=1)
        b2a = jnp.concatenate([b2l[2 * C:3 * C]] * 3
                              + [b2l[0:C]]).reshape(1, 4 * C)
        w2b = jnp.concatenate([w2l[:, 3 * C:4 * C]] * 3
                              + [w2l[:, C:2 * C]], axis=1)
        b2b = jnp.concatenate([b2l[3 * C:4 * C]] * 3
                              + [b2l[C:2 * C]]).reshape(1, 4 * C)
        wbig = jnp.zeros((4 * C, 4 * C), jnp.float32)
        for d in range(3):
            wbig = wbig.at[d * C:(d + 1) * C, d * C:(d + 1) * C].set(Wmv[l])
        wbig = wbig.at[3 * C:4 * C, 3 * C:4 * C].set(Wms[l])
        sv_full = None if l == 0 else jnp.concatenate(sv, axis=0)
        new_sv = []
        for h in range(NH):
            if l == 0:
                gathered = jnp.tile(embed, (1, 4))
            else:
                gathered = _sc_gather(sv_full, flat_idx[h], chunk=512)
            o_f32 = _run_layer(
                gathered, rbf[h], rh[h], sv[h],
                W1[l].astype(jnp.bfloat16), b1[l].reshape(1, RH),
                w2a.astype(jnp.bfloat16), b2a,
                w2b.astype(jnp.bfloat16), b2b, sel3, self1,
                Wg[l], bg[l].reshape(1, C),
                wbig, mrows, layer0=(l == 0))
            new_sv.append(o_f32)
        sv = new_sv

    return _run_readout(jnp.concatenate(sv, axis=0),
                        Wc1, bc1.reshape(1, 128),
                        Wc2, bc2.reshape(1, 64),
                        Wc3, bc3.reshape(1, NUM_CLASSES))


# speedup vs baseline: 1.0634x; 1.0152x over previous
import functools

import jax
import jax.numpy as jnp
from jax import lax
from jax.experimental import pallas as pl
from jax.experimental.pallas import tpu as pltpu
from jax.experimental.pallas import tpu_sc as plsc

B, P, K = 8, 2048, 16
C = 32
NUM_RBF = 32
CUTOFF = 5.0
NUM_LAYERS = 4
RH = 64
NUM_CLASSES = 10

BP = B * P
BE = BP * K
RKNN = 512
RNODE = 512
EBLK = RNODE * K


def _knn_body(pos_ref, post_ref, nbr_ref, *, b_base):
    b = pl.program_id(0) + b_base
    i = pl.program_id(1)
    x = pos_ref[0]
    xt = post_ref[0]
    d2 = jnp.zeros((RKNN, P), jnp.float32)
    for d in range(3):
        diff = x[:, d:d + 1] - xt[d:d + 1, :]
        d2 = d2 + diff * diff
    rows = lax.broadcasted_iota(jnp.int32, (RKNN, P), 0) + i * RKNN
    cols = lax.broadcasted_iota(jnp.int32, (RKNN, P), 1)
    d2 = jnp.where(rows == cols, d2 + 1e9, d2)
    idx_cols = []
    for _ in range(K):
        m = jnp.min(d2, axis=1, keepdims=True)
        cand = jnp.where(d2 == m, cols, jnp.int32(P))
        idx = jnp.min(cand, axis=1, keepdims=True)
        idx_cols.append(idx)
        d2 = jnp.where(cols == idx, jnp.float32(3e38), d2)
    nbr = jnp.concatenate(idx_cols, axis=1)
    nbr_ref[0] = nbr + b * P


def _run_knn(batch_h, b_base):
    nb = batch_h.shape[0]
    post = jnp.transpose(batch_h, (0, 2, 1))
    return pl.pallas_call(
        functools.partial(_knn_body, b_base=b_base),
        grid=(nb, P // RKNN),
        in_specs=[
            pl.BlockSpec((1, RKNN, 3), lambda b, i: (b, i, 0)),
            pl.BlockSpec((1, 3, P), lambda b, i: (b, 0, 0)),
        ],
        out_specs=pl.BlockSpec((1, RKNN, K), lambda b, i: (b, i, 0)),
        out_shape=jax.ShapeDtypeStruct((nb, P, K), jnp.int32),
    )(batch_h, post)


_SC_NC = 2
_SC_NS = 16
_NW = _SC_NC * _SC_NS


def _sc_gather(table, idx, chunk):
    n = idx.shape[0]
    d = table.shape[1]
    dt = table.dtype
    npw = n // _NW
    nchunks = npw // chunk
    mesh = plsc.VectorSubcoreMesh(core_axis_name="c", subcore_axis_name="s")

    @functools.partial(
        pl.kernel,
        mesh=mesh,
        out_type=jax.ShapeDtypeStruct((n, d), dt),
        scratch_types=[
            pltpu.VMEM((chunk,), jnp.int32),
            pltpu.VMEM((chunk, d), dt),
            pltpu.SemaphoreType.DMA,
        ],
    )
    def gather_k(table_hbm, idx_hbm, out_hbm, idx_v, rows_v, sem):
        wid = lax.axis_index("s") * _SC_NC + lax.axis_index("c")
        base = wid * npw

        def body(ci, carry):
            off = base + ci * chunk
            pltpu.sync_copy(idx_hbm.at[pl.ds(off, chunk)], idx_v)
            pltpu.async_copy(table_hbm.at[idx_v], rows_v, sem).wait()
            pltpu.sync_copy(rows_v, out_hbm.at[pl.ds(off, chunk)])
            return carry

        lax.fori_loop(0, nchunks, body, 0)

    return gather_k(table, idx)


def _edge_body(ps_ref, pd_ref, centers_ref, rbf_ref, rh_ref):
    ps = ps_ref[...]
    pd = pd_ref[...]
    relx = ps[:, 0:1] - pd[:, 0:1]
    rely = ps[:, 1:2] - pd[:, 1:2]
    relz = ps[:, 2:3] - pd[:, 2:3]
    r = jnp.sqrt(relx * relx + rely * rely + relz * relz + 1e-12)
    rhx = relx / r
    rhy = rely / r
    rhz = relz / r
    centers = centers_ref[...]
    gamma = jnp.float32(NUM_RBF / CUTOFF)
    t = r - centers
    rbf = jnp.exp(-gamma * (t * t))
    x = jnp.clip(r * jnp.float32(1.0 / CUTOFF), 0.0, 1.0)
    u = jnp.float32(jnp.pi) * (x - 0.5)
    u2 = u * u
    sinu = u * (1.0 + u2 * (jnp.float32(-1.0 / 6.0)
                + u2 * (jnp.float32(1.0 / 120.0)
                + u2 * (jnp.float32(-1.0 / 5040.0)
                + u2 * jnp.float32(1.0 / 362880.0)))))
    env = jnp.where(x >= 1.0, 0.0, 0.5 * (1.0 - sinu))
    rbf_ref[...] = rbf * env
    one = jnp.ones_like(rhx)
    rh_ref[...] = jnp.concatenate(
        [rhx, rhy, rhz, one, jnp.zeros((EBLK, 4), jnp.float32)], axis=1)


def _run_edges(pos_src, pos_dst, centers):
    ne = pos_src.shape[0]
    return pl.pallas_call(
        _edge_body,
        grid=(ne // EBLK,),
        in_specs=[
            pl.BlockSpec((EBLK, 128), lambda g: (g, 0)),
            pl.BlockSpec((EBLK, 4), lambda g: (g, 0)),
            pl.BlockSpec((1, NUM_RBF), lambda g: (0, 0)),
        ],
        out_specs=[
            pl.BlockSpec((EBLK, NUM_RBF), lambda g: (g, 0)),
            pl.BlockSpec((EBLK, 8), lambda g: (g, 0)),
        ],
        out_shape=[
            jax.ShapeDtypeStruct((ne, NUM_RBF), jnp.float32),
            jax.ShapeDtypeStruct((ne, 8), jnp.float32),
        ],
    )(pos_src, pos_dst, centers)


def _silu(x):
    return x * (1.0 / (1.0 + jnp.exp(-x)))


def _layer_body(g_ref, rbf_ref, rh_ref, old_ref,
                w1_ref, b1_ref, w2a_ref, b2a_ref, w2b_ref, b2b_ref,
                sel3_ref, self1_ref, wg_ref, bg_ref, wbig_ref,
                mrow1_ref, mrow02_ref, mrow3_ref, out_ref, *, layer0):
    rbf = rbf_ref[...]
    h = _silu(jnp.dot(rbf.astype(jnp.bfloat16), w1_ref[...],
                      preferred_element_type=jnp.float32) + b1_ref[...])
    hb = h.astype(jnp.bfloat16)
    a = jnp.dot(hb, w2a_ref[...],
                preferred_element_type=jnp.float32) + b2a_ref[...]
    rh = rh_ref[...]
    r3 = jnp.dot(rh, sel3_ref[...], preferred_element_type=jnp.float32)
    rfull = jnp.dot(rh, self1_ref[...], preferred_element_type=jnp.float32)
    m1 = mrow1_ref[...] != 0.0
    m02 = mrow02_ref[...] != 0.0
    m3 = mrow3_ref[...] != 0.0
    if layer0:
        s4 = jnp.broadcast_to(g_ref[...], (EBLK, 4 * C))
        msg = a * s4 * rfull
    else:
        g = g_ref[...].astype(jnp.float32)
        bt = jnp.dot(hb, w2b_ref[...],
                     preferred_element_type=jnp.float32) + b2b_ref[...]
        t = g * r3
        t2 = t + jnp.roll(t, -C, axis=1)
        t4 = t2 + jnp.roll(t2, -2 * C, axis=1)
        ghat = jnp.where(m3, t4, g)
        x1 = jnp.where(m1, jnp.roll(g, -2 * C, axis=1), g)
        s4 = jnp.where(m02, jnp.roll(x1, -C, axis=1), x1)
        msg = a * s4 * rfull + bt * ghat
    agg = jnp.sum(msg.reshape(RNODE, K, 4 * C), axis=1) * jnp.float32(1.0 / K)
    a_s = agg[:, 3 * C:4 * C]
    gate = 1.0 / (1.0 + jnp.exp(-(jnp.dot(a_s, wg_ref[...],
                                          preferred_element_type=jnp.float32)
                                  + bg_ref[...])))
    x = jnp.concatenate(
        [agg[:, 0:C] * gate, agg[:, C:2 * C] * gate, agg[:, 2 * C:3 * C] * gate,
         _silu(a_s)], axis=1)
    z = jnp.dot(x, wbig_ref[...], preferred_element_type=jnp.float32)
    out_ref[...] = old_ref[...] + z


def _run_layer(gathered, rbf, rh, sv_old, w1, b1, w2a, b2a, w2b, b2b,
               sel3, self1, wg, bg, wbig, mrows, layer0=False):
    full = lambda r, c: pl.BlockSpec((r, c), lambda g: (0, 0))
    g_spec = (full(1, 4 * C) if layer0
              else pl.BlockSpec((EBLK, 4 * C), lambda g: (g, 0)))
    nn = sv_old.shape[0]
    return pl.pallas_call(
        functools.partial(_layer_body, layer0=layer0),
        grid=(nn // RNODE,),
        in_specs=[
            g_spec,
            pl.BlockSpec((EBLK, NUM_RBF), lambda g: (g, 0)),
            pl.BlockSpec((EBLK, 8), lambda g: (g, 0)),
            pl.BlockSpec((RNODE, 4 * C), lambda g: (g, 0)),
            full(NUM_RBF, RH), full(1, RH),
            full(RH, 4 * C), full(1, 4 * C), full(RH, 4 * C), full(1, 4 * C),
            full(8, 4 * C), full(8, 4 * C),
            full(C, C), full(1, C), full(4 * C, 4 * C),
            full(1, 4 * C), full(1, 4 * C), full(1, 4 * C),
        ],
        out_specs=pl.BlockSpec((RNODE, 4 * C), lambda g: (g, 0)),
        out_shape=jax.ShapeDtypeStruct((nn, 4 * C), jnp.float32),
    )(gathered, rbf, rh, sv_old, w1, b1, w2a, b2a, w2b, b2b,
      sel3, self1, wg, bg, wbig, mrows[0], mrows[1], mrows[2])


def _readout_body(sv_ref, wc1_ref, bc1_ref, wc2_ref, bc2_ref, wc3_ref, bc3_ref,
                  out_ref):
    s = sv_ref[:, 3 * C:4 * C]
    pooled = jnp.sum(s.reshape(B, P, C), axis=1) * jnp.float32(1.0 / P)
    h = _silu(jnp.dot(pooled, wc1_ref[...],
                      preferred_element_type=jnp.float32) + bc1_ref[...])
    h = _silu(jnp.dot(h, wc2_ref[...],
                      preferred_element_type=jnp.float32) + bc2_ref[...])
    out_ref[...] = jnp.dot(h, wc3_ref[...],
                           preferred_element_type=jnp.float32) + bc3_ref[...]


def _run_readout(sv, wc1, bc1, wc2, bc2, wc3, bc3):
    return pl.pallas_call(
        _readout_body,
        out_shape=jax.ShapeDtypeStruct((B, NUM_CLASSES), jnp.float32),
    )(sv, wc1, bc1, wc2, bc2, wc3, bc3)


def kernel(batch, embed_w, W1, b1, W2, b2, Wg, bg, Wms, Wmv,
           Wc1, bc1, Wc2, bc2, Wc3, bc3):
    batch = batch.astype(jnp.float32)
    NH = 4
    BH = B // NH
    BPH, BEH = BH * P, BH * P * K

    pos_flat = batch.reshape(BP, 3)
    pos_pad = jnp.concatenate(
        [pos_flat, jnp.zeros((BP, 125), jnp.float32)], axis=1)
    centers = jnp.linspace(0.0, CUTOFF, NUM_RBF,
                           dtype=jnp.float32).reshape(1, NUM_RBF)

    flat_idx, rbf, rh = [], [], []
    for h in range(NH):
        nbr_h = _run_knn(batch[h * BH:(h + 1) * BH], h * BH)
        flat_idx.append(nbr_h.reshape(BEH))
    for h in range(NH):
        pos_src_h = _sc_gather(pos_pad, flat_idx[h], chunk=512)
        pd_h = pos_flat[h * BPH:(h + 1) * BPH]
        pos_dst_h = jnp.broadcast_to(
            pd_h[:, None, :], (BPH, K, 3)).reshape(BEH, 3)
        pos_dst_h = jnp.concatenate(
            [pos_dst_h, jnp.zeros((BEH, 1), jnp.float32)], axis=1)
        rbf_h, rh_h = _run_edges(pos_src_h, pos_dst_h, centers)
        rbf.append(rbf_h)
        rh.append(rh_h)

    embed = embed_w.astype(jnp.float32)
    sv = [jnp.concatenate(
        [jnp.zeros((BPH, 3 * C), jnp.float32),
         jnp.broadcast_to(embed, (BPH, C))], axis=1) for _ in range(NH)]

    lane = jnp.arange(4 * C, dtype=jnp.int32)
    plane = lane // C
    mrow1 = (plane == 1).astype(jnp.float32).reshape(1, 4 * C)
    mrow02 = ((plane == 0) | (plane == 2)).astype(jnp.float32).reshape(1, 4 * C)
    mrow3 = (plane == 3).astype(jnp.float32).reshape(1, 4 * C)
    mrows = (mrow1, mrow02, mrow3)
    sel3 = jnp.zeros((8, 4 * C), jnp.float32)
    for d in range(3):
        sel3 = sel3.at[d, d * C:(d + 1) * C].set(1.0)
    self1 = sel3.at[3, 3 * C:4 * C].set(1.0)

    for l in range(NUM_LAYERS):
        w2l = W2[l]
        b2l = b2[l]
        w2a = jnp.concatenate([w2l[:, 2 * C:3 * C]] * 3 + [w2l[:, 0:C]], axis=1)
        b2a = jnp.concatenate([b2l[2 * C:3 * C]] * 3
                              + [b2l[0:C]]).reshape(1, 4 * C)
        w2b = jnp.concatenate([w2l[:, 3 * C:4 * C]] * 3
                              + [w2l[:, C:2 * C]], axis=1)
        b2b = jnp.concatenate([b2l[3 * C:4 * C]] * 3
                              + [b2l[C:2 * C]]).reshape(1, 4 * C)
        wbig = jnp.zeros((4 * C, 4 * C), jnp.float32)
        for d in range(3):
            wbig = wbig.at[d * C:(d + 1) * C, d * C:(d + 1) * C].set(Wmv[l])
        wbig = wbig.at[3 * C:4 * C, 3 * C:4 * C].set(Wms[l])
        sv_full = None if l == 0 else jnp.concatenate(sv, axis=0)
        new_sv = []
        for h in range(NH):
            if l == 0:
                gathered = jnp.tile(embed, (1, 4))
            else:
                gathered = _sc_gather(sv_full, flat_idx[h], chunk=512)
            o_f32 = _run_layer(
                gathered, rbf[h], rh[h], sv[h],
                W1[l].astype(jnp.bfloat16), b1[l].reshape(1, RH),
                w2a.astype(jnp.bfloat16), b2a,
                w2b.astype(jnp.bfloat16), b2b, sel3, self1,
                Wg[l], bg[l].reshape(1, C),
                wbig, mrows, layer0=(l == 0))
            new_sv.append(o_f32)
        sv = new_sv

    return _run_readout(jnp.concatenate(sv, axis=0),
                        Wc1, bc1.reshape(1, 128),
                        Wc2, bc2.reshape(1, 64),
                        Wc3, bc3.reshape(1, NUM_CLASSES))
